# Initial kernel scaffold; baseline (speedup 1.0000x reference)
import jax
import jax.numpy as jnp
from jax.experimental import pallas as pl


def kernel(source_features, source_labels, target_features):
    raise NotImplementedError("write your pallas kernel here")


# speedup vs baseline: 197.5446x; 197.5446x over previous
import jax
import jax.numpy as jnp
from jax.experimental import pallas as pl
from jax.experimental.pallas import tpu as pltpu

RANKING_K = 5
TOP_RANKED_N = 1024
TOP_N_SIM = 7
TAU = 0.05
NUM_CLASSES = 65

N_SRC = 8192
N_TGT = 2048
FEAT = 1024
ROW_BLK = 1024
COL_BLK = 256
COL_TILE = 128
N_TILES = N_TGT // COL_TILE
C_PAD = 128

NEG = -3.0


def _normalize_body(x_ref, o_ref):
    x = x_ref[...]
    n2 = jnp.sum(x * x, axis=1, keepdims=True)
    o_ref[...] = x / jnp.maximum(jnp.sqrt(n2), 1e-12)


def _onehot_body(lab_ref, o_ref):
    lab = lab_ref[...]
    classes = jax.lax.broadcasted_iota(jnp.int32, (N_SRC, C_PAD), 1)
    o_ref[...] = (lab == classes).astype(jnp.float32)


def _matmul_body(s_ref, t_ref, o_ref):
    o_ref[...] = jax.lax.dot_general(
        s_ref[...], t_ref[...], (((1,), (1,)), ((), ())),
        preferred_element_type=jnp.float32,
        precision=jax.lax.Precision.HIGHEST,
    )


def _reduce_body(sim_ref, oh_ref, rank_ref, con_ref, work_ref, mask_ref):
    sim = sim_ref[...]
    rows = jax.lax.broadcasted_iota(jnp.int32, (N_SRC, COL_TILE), 0)

    work_ref[...] = sim
    mask_ref[...] = jnp.zeros((N_SRC, COL_TILE), jnp.float32)
    top1 = None
    for k in range(TOP_N_SIM):
        w = work_ref[...]
        m = jnp.max(w, axis=0, keepdims=True)
        if k == 0:
            top1 = m
        idx = jnp.min(jnp.where(w == m, rows, N_SRC), axis=0, keepdims=True)
        hit = rows == idx
        work_ref[...] = jnp.where(hit, NEG, w)
        mask_ref[...] = mask_ref[...] + hit.astype(jnp.float32)

    onehot_l = oh_ref[...]
    counts = jax.lax.dot_general(
        mask_ref[...], onehot_l, (((0,), (0,)), ((), ())),
        preferred_element_type=jnp.float32,
        precision=jax.lax.Precision.HIGHEST,
    )
    cmax = jnp.max(counts, axis=1, keepdims=True)
    classes_ct = jax.lax.broadcasted_iota(jnp.int32, (COL_TILE, C_PAD), 1)
    assigned = jnp.min(
        jnp.where(counts == cmax, classes_ct, C_PAD), axis=1, keepdims=True
    )
    onehot_a = (assigned == classes_ct).astype(jnp.float32)

    posf = jax.lax.dot_general(
        onehot_l, onehot_a, (((1,), (1,)), ((), ())),
        preferred_element_type=jnp.float32,
        precision=jax.lax.Precision.HIGHEST,
    )
    mask_ref[...] = posf

    def top5_sum():
        tot = jnp.zeros((1, COL_TILE), jnp.float32)
        for _ in range(RANKING_K):
            w = work_ref[...]
            m = jnp.max(w, axis=0, keepdims=True)
            tot = tot + jnp.where(m > -2.0, m, 0.0)
            idx = jnp.min(jnp.where(w == m, rows, N_SRC), axis=0, keepdims=True)
            work_ref[...] = jnp.where(rows == idx, NEG, w)
        return tot

    posm = mask_ref[...] > 0.5
    work_ref[...] = jnp.where(posm, sim, NEG)
    pos_sum = top5_sum()
    work_ref[...] = jnp.where(posm, NEG, sim)
    neg_sum = top5_sum()
    rank_ref[...] = pos_sum / neg_sum

    e = jnp.exp((sim - top1) * (1.0 / TAU))
    total = jnp.sum(e, axis=0, keepdims=True)
    pos_e = jnp.sum(e * mask_ref[...], axis=0, keepdims=True)
    con_ref[...] = pos_e / total


def _loss_body(rank_ref, con_ref, loss_ref):
    r_row = rank_ref[...]
    r_col = r_row.reshape(N_TGT, 1)
    j_row = jax.lax.broadcasted_iota(jnp.int32, (1, N_TGT), 1)
    i_col = jax.lax.broadcasted_iota(jnp.int32, (N_TGT, 1), 0)
    beats = jnp.logical_or(
        r_row > r_col, jnp.logical_and(r_row == r_col, j_row < i_col)
    )
    nbeats = jnp.sum(beats.astype(jnp.float32), axis=1, keepdims=True)
    sel = (nbeats < TOP_RANKED_N).astype(jnp.float32)
    c = con_ref[...].reshape(N_TGT, 1)
    loss = -jnp.sum(sel * jnp.log(c + 1e-6), keepdims=True) / TOP_RANKED_N
    loss_ref[...] = loss.reshape(1, 1)


def kernel(source_features, source_labels, target_features):
    s_norm = pl.pallas_call(
        _normalize_body,
        grid=(8,),
        in_specs=[pl.BlockSpec((N_SRC // 8, FEAT), lambda i: (i, 0))],
        out_specs=pl.BlockSpec((N_SRC // 8, FEAT), lambda i: (i, 0)),
        out_shape=jax.ShapeDtypeStruct((N_SRC, FEAT), jnp.float32),
    )(source_features)

    t_norm = pl.pallas_call(
        _normalize_body,
        grid=(2,),
        in_specs=[pl.BlockSpec((N_TGT // 2, FEAT), lambda i: (i, 0))],
        out_specs=pl.BlockSpec((N_TGT // 2, FEAT), lambda i: (i, 0)),
        out_shape=jax.ShapeDtypeStruct((N_TGT, FEAT), jnp.float32),
    )(target_features)

    lab2 = source_labels.reshape(N_SRC, 1).astype(jnp.int32)
    onehot_l = pl.pallas_call(
        _onehot_body,
        in_specs=[pl.BlockSpec((N_SRC, 1), lambda: (0, 0))],
        out_specs=pl.BlockSpec((N_SRC, C_PAD), lambda: (0, 0)),
        out_shape=jax.ShapeDtypeStruct((N_SRC, C_PAD), jnp.float32),
    )(lab2)

    sim = pl.pallas_call(
        _matmul_body,
        grid=(N_SRC // ROW_BLK, N_TGT // COL_BLK),
        in_specs=[
            pl.BlockSpec((ROW_BLK, FEAT), lambda i, j: (i, 0)),
            pl.BlockSpec((COL_BLK, FEAT), lambda i, j: (j, 0)),
        ],
        out_specs=pl.BlockSpec((ROW_BLK, COL_BLK), lambda i, j: (i, j)),
        out_shape=jax.ShapeDtypeStruct((N_SRC, N_TGT), jnp.float32),
        compiler_params=pltpu.CompilerParams(
            dimension_semantics=("parallel", "parallel"),
        ),
    )(s_norm, t_norm)

    ranking, contrast = pl.pallas_call(
        _reduce_body,
        grid=(N_TILES,),
        in_specs=[
            pl.BlockSpec((N_SRC, COL_TILE), lambda i: (0, i)),
            pl.BlockSpec((N_SRC, C_PAD), lambda i: (0, 0)),
        ],
        out_specs=[
            pl.BlockSpec((1, COL_TILE), lambda i: (0, i)),
            pl.BlockSpec((1, COL_TILE), lambda i: (0, i)),
        ],
        out_shape=[
            jax.ShapeDtypeStruct((1, N_TGT), jnp.float32),
            jax.ShapeDtypeStruct((1, N_TGT), jnp.float32),
        ],
        scratch_shapes=[
            pltpu.VMEM((N_SRC, COL_TILE), jnp.float32),
            pltpu.VMEM((N_SRC, COL_TILE), jnp.float32),
        ],
        compiler_params=pltpu.CompilerParams(
            dimension_semantics=("arbitrary",),
        ),
    )(sim, onehot_l)

    loss = pl.pallas_call(
        _loss_body,
        in_specs=[
            pl.BlockSpec((1, N_TGT), lambda: (0, 0)),
            pl.BlockSpec((1, N_TGT), lambda: (0, 0)),
        ],
        out_specs=pl.BlockSpec((1, 1), lambda: (0, 0)),
        out_shape=jax.ShapeDtypeStruct((1, 1), jnp.float32),
    )(ranking, contrast)

    return loss[0, 0]
